# baseline (speedup 1.0000x reference)
import functools

import jax
import jax.numpy as jnp
from jax import lax
from jax.experimental import pallas as pl
from jax.experimental.pallas import tpu as pltpu
from jax.experimental.pallas import tpu_sc as plsc

NUM_CORES = 2
NUM_SUBCORES = 16
LANES = 16
NW = NUM_CORES * NUM_SUBCORES

BATCH = 16384
DIM = 64
BPW = BATCH // NW
CHUNK = 128
NCHUNK = BPW // CHUNK


def _sc_body(t_idx_hbm, c_idx_hbm, table_hbm, out_hbm,
             t_idx_v, c_idx_v, t_rows, c_rows, out_v, sem):
    w = lax.axis_index("s") * NUM_CORES + lax.axis_index("c")

    pltpu.sync_copy(t_idx_hbm.at[pl.ds(w * NCHUNK, NCHUNK)], t_idx_v)
    pltpu.sync_copy(c_idx_hbm.at[pl.ds(w * NCHUNK, NCHUNK)], c_idx_v)

    copies = []
    for j in range(NCHUNK):
        copies.append(pltpu.async_copy(
            table_hbm.at[t_idx_v.at[j]], t_rows.at[pl.ds(j * CHUNK, CHUNK)], sem))
        copies.append(pltpu.async_copy(
            table_hbm.at[c_idx_v.at[j]], c_rows.at[pl.ds(j * CHUNK, CHUNK)], sem))
    for c in copies:
        c.wait()

    def block(b, carry):
        rows16 = b * LANES + lax.iota(jnp.int32, LANES)
        acc = jnp.zeros((LANES,), jnp.float32)
        for d in range(DIM):
            dcol = jnp.full((LANES,), d, jnp.int32)
            tv = plsc.load_gather(t_rows, [rows16, dcol])
            cv = plsc.load_gather(c_rows, [rows16, dcol])
            acc = acc + tv * cv
        out_v[pl.ds(b * LANES, LANES)] = 1.0 / (1.0 + jnp.exp(-acc))
        return carry

    lax.fori_loop(0, BPW // LANES, block, 0)

    pltpu.sync_copy(out_v, out_hbm.at[pl.ds(w * BPW, BPW)])


@functools.cache
def _sc_call():
    return functools.partial(
        pl.kernel,
        out_type=jax.ShapeDtypeStruct((BATCH,), jnp.float32),
        mesh=plsc.VectorSubcoreMesh(
            core_axis_name="c", subcore_axis_name="s",
            num_cores=NUM_CORES, num_subcores=NUM_SUBCORES),
        scratch_types=[
            pltpu.VMEM((NCHUNK, CHUNK), jnp.int32),
            pltpu.VMEM((NCHUNK, CHUNK), jnp.int32),
            pltpu.VMEM((BPW, DIM), jnp.float32),
            pltpu.VMEM((BPW, DIM), jnp.float32),
            pltpu.VMEM((BPW,), jnp.float32),
            pltpu.SemaphoreType.DMA,
        ],
        compiler_params=pltpu.CompilerParams(
            needs_layout_passes=False, use_tc_tiling_on_sc=False),
    )(_sc_body)


@jax.jit
def kernel(target_items, context_items, table):
    t2 = target_items.astype(jnp.int32).reshape(NW * NCHUNK, CHUNK)
    c2 = context_items.astype(jnp.int32).reshape(NW * NCHUNK, CHUNK)
    return _sc_call()(t2, c2, table)


# speedup vs baseline: 1.6394x; 1.6394x over previous
import functools

import jax
import jax.numpy as jnp
from jax import lax
from jax.experimental import pallas as pl
from jax.experimental.pallas import tpu as pltpu
from jax.experimental.pallas import tpu_sc as plsc

NUM_CORES = 2
NUM_SUBCORES = 16
LANES = 16
NW = NUM_CORES * NUM_SUBCORES

BATCH = 16384
DIM = 64
BPW = BATCH // NW
CH = 256
NCH = BPW // CH


def _sc_body(t_idx_hbm, c_idx_hbm, table_hbm, out_hbm,
             t_idx_v, c_idx_v, t_rows, c_rows, out_v, sem):
    w = lax.axis_index("s") * NUM_CORES + lax.axis_index("c")
    base = w * BPW

    pltpu.sync_copy(t_idx_hbm.at[pl.ds(base, BPW)], t_idx_v)
    pltpu.sync_copy(c_idx_hbm.at[pl.ds(base, BPW)], c_idx_v)

    def run_pass(p, carry):
        off = p * CH

        def fire(g, c2):
            tvec = t_idx_v[pl.ds(off + g * LANES, LANES)]
            cvec = c_idx_v[pl.ds(off + g * LANES, LANES)]
            for j in range(LANES):
                pltpu.async_copy(
                    table_hbm.at[pl.ds(tvec[j], 1)],
                    t_rows.at[pl.ds(g * LANES + j, 1)], sem)
                pltpu.async_copy(
                    table_hbm.at[pl.ds(cvec[j], 1)],
                    c_rows.at[pl.ds(g * LANES + j, 1)], sem)
            return c2

        lax.fori_loop(0, CH // LANES, fire, 0)

        def drain(i, c2):
            pltpu.make_async_copy(
                table_hbm.at[pl.ds(0, 1)], t_rows.at[pl.ds(0, 1)], sem).wait()
            return c2

        lax.fori_loop(0, 2 * CH, drain, 0)

        def block(b, c2):
            rows16 = b * LANES + lax.iota(jnp.int32, LANES)
            acc = jnp.zeros((LANES,), jnp.float32)
            for d in range(DIM):
                dcol = jnp.full((LANES,), d, jnp.int32)
                tv = plsc.load_gather(t_rows, [rows16, dcol])
                cv = plsc.load_gather(c_rows, [rows16, dcol])
                acc = acc + tv * cv
            out_v[pl.ds(off + b * LANES, LANES)] = 1.0 / (1.0 + jnp.exp(-acc))
            return c2

        lax.fori_loop(0, CH // LANES, block, 0)
        return carry

    lax.fori_loop(0, NCH, run_pass, 0)

    pltpu.sync_copy(out_v, out_hbm.at[pl.ds(base, BPW)])


@functools.cache
def _sc_call():
    return functools.partial(
        pl.kernel,
        out_type=jax.ShapeDtypeStruct((BATCH,), jnp.float32),
        mesh=plsc.VectorSubcoreMesh(
            core_axis_name="c", subcore_axis_name="s",
            num_cores=NUM_CORES, num_subcores=NUM_SUBCORES),
        scratch_types=[
            pltpu.VMEM((BPW,), jnp.int32),
            pltpu.VMEM((BPW,), jnp.int32),
            pltpu.VMEM((CH, DIM), jnp.float32),
            pltpu.VMEM((CH, DIM), jnp.float32),
            pltpu.VMEM((BPW,), jnp.float32),
            pltpu.SemaphoreType.DMA,
        ],
        compiler_params=pltpu.CompilerParams(needs_layout_passes=False),
    )(_sc_body)


@jax.jit
def kernel(target_items, context_items, table):
    t = target_items.astype(jnp.int32)
    c = context_items.astype(jnp.int32)
    return _sc_call()(t, c, table)
